# baseline (speedup 1.0000x reference)
import functools

import jax
import jax.numpy as jnp
from jax import lax
from jax.experimental import pallas as pl
from jax.experimental.pallas import tpu as pltpu
from jax.experimental.pallas import tpu_sc as plsc

N = 10000
E = 320000
D = 128
DW = 16
NC = 2
NS = 16
L = 16
NW = NC * NS
K2 = 48
NCHP = 210
EPW = NCHP * K2
E_PAD = NW * EPW
N_PAD = 10016
RPT = N_PAD // NS

_SC_MESH = plsc.VectorSubcoreMesh(core_axis_name="c", subcore_axis_name="s",
                                  num_cores=NC, num_subcores=NS)
_SC_PARAMS = pltpu.CompilerParams(use_tc_tiling_on_sc=False,
                                  needs_layout_passes=False)


@functools.partial(
    pl.kernel,
    out_type=[
        jax.ShapeDtypeStruct((NC, N_PAD, D), jnp.float32),
        jax.ShapeDtypeStruct((NC, N_PAD, DW), jnp.float32),
    ],
    mesh=_SC_MESH,
    compiler_params=_SC_PARAMS,
    scratch_types=[
        pltpu.VMEM((N_PAD,), jnp.float32),
        pltpu.VMEM((N_PAD,), jnp.float32),
        pltpu.VMEM((4, 2, K2), jnp.int32),
        pltpu.VMEM((K2,), jnp.float32),
        pltpu.VMEM((K2, D), jnp.float32),
        pltpu.VMEM((K2, D), jnp.float32),
        pltpu.VMEM((K2, D), jnp.float32),
        pltpu.VMEM((K2, DW), jnp.float32),
        pltpu.VMEM((K2, DW), jnp.float32),
        pltpu.VMEM_SHARED((N_PAD, D), jnp.float32),
        pltpu.VMEM_SHARED((N_PAD, DW), jnp.float32),
        pltpu.SemaphoreType.DMA,
        pltpu.SemaphoreType.DMA,
        pltpu.SemaphoreType.DMA,
        pltpu.SemaphoreType.DMA,
        pltpu.SemaphoreType.DMA,
        pltpu.SemaphoreType.DMA,
        pltpu.SemaphoreType.DMA,
        pltpu.SemaphoreType.DMA,
        pltpu.SemaphoreType.DMA,
    ],
)
def _sc_layer_edges(h_hbm, asrc_hbm, adst_hbm, pk_hbm, num_hbm, den_hbm,
                    asrc_v, adst_v, pk, eeb, g0, g1, g2, w0, w1,
                    acc_n, acc_d, psem0, psem1, psem2, psem3,
                    gsem0, gsem1, gsem2, ssem0, ssem1):
    cid = lax.axis_index("c")
    sid = lax.axis_index("s")
    wid = cid * NS + sid
    psems = (psem0, psem1, psem2, psem3)
    gsems = (gsem0, gsem1, gsem2)
    ssems = (ssem0, ssem1)
    gbufs = (g0, g1, g2)
    wbufs = (w0, w1)

    pltpu.sync_copy(asrc_hbm, asrc_v)
    pltpu.sync_copy(adst_hbm, adst_v)

    def _z(r, _):
        for j in range(D // L):
            g0[r, pl.ds(j * L, L)] = jnp.zeros((L,), jnp.float32)
        w0[r, pl.ds(0, L)] = jnp.zeros((L,), jnp.float32)
        w1[r, pl.ds(0, L)] = jnp.zeros((L,), jnp.float32)
        return 0

    lax.fori_loop(0, K2, _z, 0)
    base = sid * RPT
    nfull = RPT // K2
    rem = RPT - nfull * K2
    for q in range(nfull):
        pltpu.sync_copy(g0, acc_n.at[pl.ds(base + q * K2, K2)])
        pltpu.sync_copy(w0, acc_d.at[pl.ds(base + q * K2, K2)])
    pltpu.sync_copy(g0.at[pl.ds(0, rem)],
                    acc_n.at[pl.ds(base + nfull * K2, rem)])
    pltpu.sync_copy(w0.at[pl.ds(0, rem)],
                    acc_d.at[pl.ds(base + nfull * K2, rem)])
    plsc.subcore_barrier()

    unit = (lax.iota(jnp.int32, L) == 0).astype(jnp.float32)

    def _issue_pk(c, m):
        pltpu.async_copy(pk_hbm.at[wid * NCHP + c], pk.at[m], psems[m])

    def _wait_pk(c, m):
        pltpu.make_async_copy(pk_hbm.at[wid * NCHP + c], pk.at[m],
                              psems[m]).wait()

    def _issue_gather(m, n):
        pltpu.async_copy(h_hbm.at[pk.at[m, 0]], gbufs[n], gsems[n])

    def _wait_gather(m, n):
        pltpu.make_async_copy(h_hbm.at[pk.at[m, 0]], gbufs[n],
                              gsems[n]).wait()

    def _wait_scatter(m, n, v):
        pltpu.make_async_copy(gbufs[n], acc_n.at[pk.at[m, 1]],
                              ssems[v]).wait()
        pltpu.make_async_copy(wbufs[v], acc_d.at[pk.at[m, 1]],
                              ssems[v]).wait()

    _issue_pk(0, 0)
    _issue_pk(1, 1)
    _issue_pk(2, 2)
    _wait_pk(0, 0)
    _wait_pk(1, 1)
    _issue_gather(0, 0)
    _issue_gather(1, 1)

    def _slot(c, m, n, v):
        @pl.when(c < NCHP)
        def _():
            @pl.when(c >= 1)
            def _():
                _wait_scatter((m + 3) % 4, (n + 2) % 3, 1 - v)

            @pl.when(c + 3 < NCHP)
            def _():
                _issue_pk(c + 3, (m + 3) % 4)

            @pl.when(c + 2 < NCHP)
            def _():
                _wait_pk(c + 2, (m + 2) % 4)
                _issue_gather((m + 2) % 4, (n + 2) % 3)

            _wait_gather(m, n)

            for j in range(K2 // L):
                s16 = pk[m, 0, pl.ds(j * L, L)]
                d16 = pk[m, 1, pl.ds(j * L, L)]
                e = (plsc.load_gather(asrc_v, [s16]) +
                     plsc.load_gather(adst_v, [d16]))
                e = jnp.where(e > 0.0, e, 0.2 * e)
                eeb[pl.ds(j * L, L)] = jnp.exp(e)

            @plsc.parallel_loop(0, K2, unroll=4)
            def _row(r):
                w = plsc.load_gather(eeb, [jnp.full((L,), r, jnp.int32)])
                for jj in range(D // L):
                    gbufs[n][r, pl.ds(jj * L, L)] = (
                        gbufs[n][r, pl.ds(jj * L, L)] * w)
                wbufs[v][r, pl.ds(0, L)] = w * unit

            pltpu.async_copy(gbufs[n], acc_n.at[pk.at[m, 1]], ssems[v],
                             add=True)
            pltpu.async_copy(wbufs[v], acc_d.at[pk.at[m, 1]], ssems[v],
                             add=True)

    def _blk(i, _):
        for j in range(12):
            _slot(12 * i + j, j & 3, j % 3, j & 1)
        return 0

    lax.fori_loop(0, (NCHP + 11) // 12, _blk, 0)

    _wait_scatter((NCHP - 1) % 4, (NCHP - 1) % 3, (NCHP - 1) % 2)
    plsc.subcore_barrier()

    pltpu.sync_copy(acc_n.at[pl.ds(sid * RPT, RPT)],
                    num_hbm.at[cid, pl.ds(sid * RPT, RPT)])
    pltpu.sync_copy(acc_d.at[pl.ds(sid * RPT, RPT)],
                    den_hbm.at[cid, pl.ds(sid * RPT, RPT)])


BN = 1000


def _tc_in_body(x_ref, w_ref, aa_ref, h_ref, al_ref):
    h = jnp.dot(x_ref[...], w_ref[...], preferred_element_type=jnp.float32)
    h_ref[...] = h
    al_ref[...] = jnp.dot(h, aa_ref[...], preferred_element_type=jnp.float32)


def _tc_mid_body(num_ref, den_ref, b_ref, w_ref, aa_ref, h_ref, al_ref):
    s = num_ref[0] + num_ref[1]
    den = den_ref[0][:, 0:1] + den_ref[1][:, 0:1]
    z = jnp.maximum(s / (den + 1e-16) + b_ref[...], 0.0)
    h = jnp.dot(z, w_ref[...], preferred_element_type=jnp.float32)
    h_ref[...] = h
    al_ref[...] = jnp.dot(h, aa_ref[...], preferred_element_type=jnp.float32)


def _tc_out_body(num_ref, den_ref, b_ref, o_ref):
    s = num_ref[0] + num_ref[1]
    den = den_ref[0][:, 0:1] + den_ref[1][:, 0:1]
    o_ref[...] = jnp.maximum(s / (den + 1e-16) + b_ref[...], 0.0)


_tc_in = pl.pallas_call(
    _tc_in_body,
    grid=(N // BN,),
    in_specs=[
        pl.BlockSpec((BN, D), lambda i: (i, 0)),
        pl.BlockSpec((D, D), lambda i: (0, 0)),
        pl.BlockSpec((D, D), lambda i: (0, 0)),
    ],
    out_specs=[
        pl.BlockSpec((BN, D), lambda i: (i, 0)),
        pl.BlockSpec((BN, D), lambda i: (i, 0)),
    ],
    out_shape=[
        jax.ShapeDtypeStruct((N, D), jnp.float32),
        jax.ShapeDtypeStruct((N, D), jnp.float32),
    ],
)

_tc_mid = pl.pallas_call(
    _tc_mid_body,
    grid=(N // BN,),
    in_specs=[
        pl.BlockSpec((NC, BN, D), lambda i: (0, i, 0)),
        pl.BlockSpec((NC, BN, DW), lambda i: (0, i, 0)),
        pl.BlockSpec((1, D), lambda i: (0, 0)),
        pl.BlockSpec((D, D), lambda i: (0, 0)),
        pl.BlockSpec((D, D), lambda i: (0, 0)),
    ],
    out_specs=[
        pl.BlockSpec((BN, D), lambda i: (i, 0)),
        pl.BlockSpec((BN, D), lambda i: (i, 0)),
    ],
    out_shape=[
        jax.ShapeDtypeStruct((N, D), jnp.float32),
        jax.ShapeDtypeStruct((N, D), jnp.float32),
    ],
)

_tc_out = pl.pallas_call(
    _tc_out_body,
    grid=(N // BN,),
    in_specs=[
        pl.BlockSpec((NC, BN, D), lambda i: (0, i, 0)),
        pl.BlockSpec((NC, BN, DW), lambda i: (0, i, 0)),
        pl.BlockSpec((1, D), lambda i: (0, 0)),
    ],
    out_specs=pl.BlockSpec((BN, D), lambda i: (i, 0)),
    out_shape=jax.ShapeDtypeStruct((N, D), jnp.float32),
)


def kernel(x, edge_index, W1, a_src1, a_dst1, b1, W2, a_src2, a_dst2, b2):
    ei = edge_index.astype(jnp.int32)
    pad = E_PAD - E
    src1d = jnp.concatenate([ei[0], jnp.zeros((pad,), jnp.int32)])
    dst1d = jnp.concatenate(
        [ei[1], N + (jnp.arange(pad, dtype=jnp.int32) % (N_PAD - N))])
    pk_hbm = jnp.stack(
        [src1d.reshape(NW * NCHP, K2), dst1d.reshape(NW * NCHP, K2)], axis=1)

    def aa_pad(a_s, a_d):
        aa = jnp.zeros((D, D), jnp.float32)
        return aa.at[:, 0].set(a_s).at[:, 1].set(a_d)

    def alpads(al):
        return (jnp.pad(al[:, 0], (0, N_PAD - N)),
                jnp.pad(al[:, 1], (0, N_PAD - N)))

    h1, al1 = _tc_in(x, W1, aa_pad(a_src1, a_dst1))
    as1, ad1 = alpads(al1)
    num1, den1 = _sc_layer_edges(h1, as1, ad1, pk_hbm)
    h2, al2 = _tc_mid(num1, den1, b1.reshape(1, D), W2,
                      aa_pad(a_src2, a_dst2))
    as2, ad2 = alpads(al2)
    num2, den2 = _sc_layer_edges(h2, as2, ad2, pk_hbm)
    return _tc_out(num2, den2, b2.reshape(1, D))


# speedup vs baseline: 1.1526x; 1.1526x over previous
import functools

import jax
import jax.numpy as jnp
from jax import lax
from jax.experimental import pallas as pl
from jax.experimental.pallas import tpu as pltpu
from jax.experimental.pallas import tpu_sc as plsc

N = 10000
E = 320000
D = 128
DE = 144
DW = 16
NC = 2
NS = 16
L = 16
NW = NC * NS
K2 = 64
NCHP = 157
EPW = NCHP * K2
E_PAD = NW * EPW
N_PAD = 10016
RPT = N_PAD // NS

_SC_MESH = plsc.VectorSubcoreMesh(core_axis_name="c", subcore_axis_name="s",
                                  num_cores=NC, num_subcores=NS)
_SC_PARAMS = pltpu.CompilerParams(use_tc_tiling_on_sc=False,
                                  needs_layout_passes=False)


@functools.partial(
    pl.kernel,
    out_type=jax.ShapeDtypeStruct((NW, EPW), jnp.float32),
    mesh=_SC_MESH,
    compiler_params=_SC_PARAMS,
    scratch_types=[
        pltpu.VMEM((N,), jnp.float32),
        pltpu.VMEM((N,), jnp.float32),
        pltpu.VMEM((EPW,), jnp.int32),
        pltpu.VMEM((EPW,), jnp.int32),
        pltpu.VMEM((EPW,), jnp.float32),
    ],
)
def _sc_weights(asrc_hbm, adst_hbm, src_hbm, dst_hbm, ee_hbm,
                asrc_v, adst_v, sidx_v, didx_v, ee_v):
    cid = lax.axis_index("c")
    sid = lax.axis_index("s")
    wid = cid * NS + sid
    pltpu.sync_copy(src_hbm.at[wid], sidx_v)
    pltpu.sync_copy(dst_hbm.at[wid], didx_v)
    pltpu.sync_copy(asrc_hbm, asrc_v)
    pltpu.sync_copy(adst_hbm, adst_v)

    @plsc.parallel_loop(0, EPW // L, unroll=4)
    def _e16(i):
        s16 = jnp.minimum(sidx_v[pl.ds(i * L, L)], N - 1)
        d16 = jnp.minimum(didx_v[pl.ds(i * L, L)], N - 1)
        a_s = plsc.load_gather(asrc_v, [s16])
        a_d = plsc.load_gather(adst_v, [d16])
        e = a_s + a_d
        e = jnp.where(e > 0.0, e, 0.2 * e)
        ee_v[pl.ds(i * L, L)] = jnp.exp(e)
    pltpu.sync_copy(ee_v, ee_hbm.at[wid])


@functools.partial(
    pl.kernel,
    out_type=[
        jax.ShapeDtypeStruct((NC, N_PAD, D), jnp.float32),
        jax.ShapeDtypeStruct((NC, N_PAD, DW), jnp.float32),
    ],
    mesh=_SC_MESH,
    compiler_params=_SC_PARAMS,
    scratch_types=[
        pltpu.VMEM((4, 2, K2), jnp.int32),
        pltpu.VMEM((4, K2), jnp.float32),
        pltpu.VMEM((K2, D), jnp.float32),
        pltpu.VMEM((K2, D), jnp.float32),
        pltpu.VMEM((K2, D), jnp.float32),
        pltpu.VMEM((K2, D), jnp.float32),
        pltpu.VMEM((K2, DW), jnp.float32),
        pltpu.VMEM((K2, DW), jnp.float32),
        pltpu.VMEM_SHARED((N_PAD, D), jnp.float32),
        pltpu.VMEM_SHARED((N_PAD, DW), jnp.float32),
        pltpu.SemaphoreType.DMA,
        pltpu.SemaphoreType.DMA,
        pltpu.SemaphoreType.DMA,
        pltpu.SemaphoreType.DMA,
        pltpu.SemaphoreType.DMA,
        pltpu.SemaphoreType.DMA,
        pltpu.SemaphoreType.DMA,
        pltpu.SemaphoreType.DMA,
    ],
)
def _sc_aggregate(h_hbm, pk_hbm, ee_hbm, num_hbm, den_hbm,
                  pk, eeb, g0, g1, s0, s1, w0, w1, acc_n, acc_d,
                  isem0, isem1, isem2, isem3, gsem0, gsem1, ssem0, ssem1):
    cid = lax.axis_index("c")
    sid = lax.axis_index("s")
    wid = cid * NS + sid
    isems = (isem0, isem1, isem2, isem3)
    gsems = (gsem0, gsem1)
    ssems = (ssem0, ssem1)
    gbufs = (g0, g1)
    sbufs = (s0, s1)
    wbufs = (w0, w1)

    def _z(r, _):
        for j in range(D // L):
            g0[r, pl.ds(j * L, L)] = jnp.zeros((L,), jnp.float32)
        w0[r, pl.ds(0, L)] = jnp.zeros((L,), jnp.float32)
        w1[r, pl.ds(0, L)] = jnp.zeros((L,), jnp.float32)
        return 0

    lax.fori_loop(0, K2, _z, 0)
    base = sid * RPT
    nfull = RPT // K2
    rem = RPT - nfull * K2
    for q in range(nfull):
        pltpu.sync_copy(g0, acc_n.at[pl.ds(base + q * K2, K2)])
        pltpu.sync_copy(w0, acc_d.at[pl.ds(base + q * K2, K2)])
    pltpu.sync_copy(g0.at[pl.ds(0, rem)],
                    acc_n.at[pl.ds(base + nfull * K2, rem)])
    pltpu.sync_copy(w0.at[pl.ds(0, rem)],
                    acc_d.at[pl.ds(base + nfull * K2, rem)])
    plsc.subcore_barrier()

    unit = (lax.iota(jnp.int32, L) == 0).astype(jnp.float32)

    def _issue_idx(c, m):
        eb = wid * EPW + c * K2
        pltpu.async_copy(pk_hbm.at[wid * NCHP + c], pk.at[m], isems[m])
        pltpu.async_copy(ee_hbm.at[pl.ds(eb, K2)], eeb.at[m], isems[m])

    def _wait_idx(c, m):
        eb = wid * EPW + c * K2
        pltpu.make_async_copy(pk_hbm.at[wid * NCHP + c], pk.at[m],
                              isems[m]).wait()
        pltpu.make_async_copy(ee_hbm.at[pl.ds(eb, K2)], eeb.at[m],
                              isems[m]).wait()

    def _issue_gather(m, b):
        pltpu.async_copy(h_hbm.at[pk.at[m, 0]], gbufs[b], gsems[b])

    def _wait_gather(m, b):
        pltpu.make_async_copy(h_hbm.at[pk.at[m, 0]], gbufs[b],
                              gsems[b]).wait()

    def _wait_scatter(m, b):
        pltpu.make_async_copy(sbufs[b], acc_n.at[pk.at[m, 1]],
                              ssems[b]).wait()
        pltpu.make_async_copy(wbufs[b], acc_d.at[pk.at[m, 1]],
                              ssems[b]).wait()

    _issue_idx(0, 0)
    _issue_idx(1, 1)
    _wait_idx(0, 0)
    _issue_gather(0, 0)

    def _slot(c, b, m):
        @pl.when(c < NCHP)
        def _():
            @pl.when(c + 1 < NCHP)
            def _():
                _wait_idx(c + 1, (m + 1) % 4)
                _issue_gather((m + 1) % 4, 1 - b)

            _wait_gather(m, b)

            @pl.when(c >= 2)
            def _():
                _wait_scatter((m + 2) % 4, b)

            @plsc.parallel_loop(0, K2, unroll=8)
            def _row(r):
                w = plsc.load_gather(eeb.at[m], [jnp.full((L,), r, jnp.int32)])
                for jj in range(D // L):
                    sbufs[b][r, pl.ds(jj * L, L)] = (
                        gbufs[b][r, pl.ds(jj * L, L)] * w)
                wbufs[b][r, pl.ds(0, L)] = w * unit

            pltpu.async_copy(sbufs[b], acc_n.at[pk.at[m, 1]], ssems[b],
                             add=True)
            pltpu.async_copy(wbufs[b], acc_d.at[pk.at[m, 1]], ssems[b],
                             add=True)

            @pl.when(c + 2 < NCHP)
            def _():
                _issue_idx(c + 2, (m + 2) % 4)

    def _quad(i, _):
        for b4 in range(4):
            _slot(4 * i + b4, b4 & 1, b4)
        return 0

    lax.fori_loop(0, (NCHP + 3) // 4, _quad, 0)

    _wait_scatter((NCHP - 2) % 4, (NCHP - 2) % 2)
    _wait_scatter((NCHP - 1) % 4, (NCHP - 1) % 2)
    plsc.subcore_barrier()

    pltpu.sync_copy(acc_n.at[pl.ds(sid * RPT, RPT)],
                    num_hbm.at[cid, pl.ds(sid * RPT, RPT)])
    pltpu.sync_copy(acc_d.at[pl.ds(sid * RPT, RPT)],
                    den_hbm.at[cid, pl.ds(sid * RPT, RPT)])


BN = 1000


def _tc_in_body(x_ref, w_ref, aa_ref, h_ref, al_ref):
    h = jnp.dot(x_ref[...], w_ref[...], preferred_element_type=jnp.float32)
    h_ref[...] = h
    al_ref[...] = jnp.dot(h, aa_ref[...], preferred_element_type=jnp.float32)


def _tc_mid_body(num_ref, den_ref, b_ref, w_ref, aa_ref, h_ref, al_ref):
    s = num_ref[0] + num_ref[1]
    den = den_ref[0][:, 0:1] + den_ref[1][:, 0:1]
    z = jnp.maximum(s / (den + 1e-16) + b_ref[...], 0.0)
    h = jnp.dot(z, w_ref[...], preferred_element_type=jnp.float32)
    h_ref[...] = h
    al_ref[...] = jnp.dot(h, aa_ref[...], preferred_element_type=jnp.float32)


def _tc_out_body(num_ref, den_ref, b_ref, o_ref):
    s = num_ref[0] + num_ref[1]
    den = den_ref[0][:, 0:1] + den_ref[1][:, 0:1]
    o_ref[...] = jnp.maximum(s / (den + 1e-16) + b_ref[...], 0.0)


_tc_in = pl.pallas_call(
    _tc_in_body,
    grid=(N // BN,),
    in_specs=[
        pl.BlockSpec((BN, D), lambda i: (i, 0)),
        pl.BlockSpec((D, D), lambda i: (0, 0)),
        pl.BlockSpec((D, D), lambda i: (0, 0)),
    ],
    out_specs=[
        pl.BlockSpec((BN, D), lambda i: (i, 0)),
        pl.BlockSpec((BN, D), lambda i: (i, 0)),
    ],
    out_shape=[
        jax.ShapeDtypeStruct((N, D), jnp.float32),
        jax.ShapeDtypeStruct((N, D), jnp.float32),
    ],
)

_tc_mid = pl.pallas_call(
    _tc_mid_body,
    grid=(N // BN,),
    in_specs=[
        pl.BlockSpec((NC, BN, D), lambda i: (0, i, 0)),
        pl.BlockSpec((NC, BN, DW), lambda i: (0, i, 0)),
        pl.BlockSpec((1, D), lambda i: (0, 0)),
        pl.BlockSpec((D, D), lambda i: (0, 0)),
        pl.BlockSpec((D, D), lambda i: (0, 0)),
    ],
    out_specs=[
        pl.BlockSpec((BN, D), lambda i: (i, 0)),
        pl.BlockSpec((BN, D), lambda i: (i, 0)),
    ],
    out_shape=[
        jax.ShapeDtypeStruct((N, D), jnp.float32),
        jax.ShapeDtypeStruct((N, D), jnp.float32),
    ],
)

_tc_out = pl.pallas_call(
    _tc_out_body,
    grid=(N // BN,),
    in_specs=[
        pl.BlockSpec((NC, BN, D), lambda i: (0, i, 0)),
        pl.BlockSpec((NC, BN, DW), lambda i: (0, i, 0)),
        pl.BlockSpec((1, D), lambda i: (0, 0)),
    ],
    out_specs=pl.BlockSpec((BN, D), lambda i: (i, 0)),
    out_shape=jax.ShapeDtypeStruct((N, D), jnp.float32),
)


def kernel(x, edge_index, W1, a_src1, a_dst1, b1, W2, a_src2, a_dst2, b2):
    ei = edge_index.astype(jnp.int32)
    pad = E_PAD - E
    src1d = jnp.concatenate([ei[0], jnp.zeros((pad,), jnp.int32)])
    dst1d = jnp.concatenate(
        [ei[1], N + (jnp.arange(pad, dtype=jnp.int32) % (N_PAD - N))])
    src2d = src1d.reshape(NW, EPW)
    dst2d = dst1d.reshape(NW, EPW)
    pk_hbm = jnp.stack(
        [src1d.reshape(NW * NCHP, K2), dst1d.reshape(NW * NCHP, K2)], axis=1)

    def aa_pad(a_s, a_d):
        aa = jnp.zeros((D, D), jnp.float32)
        return aa.at[:, 0].set(a_s).at[:, 1].set(a_d)

    h1, al1 = _tc_in(x, W1, aa_pad(a_src1, a_dst1))
    ee1 = _sc_weights(al1[:, 0], al1[:, 1], src2d, dst2d)
    num1, den1 = _sc_aggregate(h1, pk_hbm, ee1.reshape(-1))
    h2, al2 = _tc_mid(num1, den1, b1.reshape(1, D), W2,
                      aa_pad(a_src2, a_dst2))
    ee2 = _sc_weights(al2[:, 0], al2[:, 1], src2d, dst2d)
    num2, den2 = _sc_aggregate(h2, pk_hbm, ee2.reshape(-1))
    return _tc_out(num2, den2, b2.reshape(1, D))
